# Initial kernel scaffold; baseline (speedup 1.0000x reference)
import functools

import jax
import jax.numpy as jnp
from jax import lax
from jax.experimental import pallas as pl
from jax.experimental.pallas import tpu as pltpu
from jax.experimental.pallas import tpu_sc as plsc

_NUM_CORES = 2
_NUM_SUBCORES = 16
_NUM_WORKERS = _NUM_CORES * _NUM_SUBCORES
_IDX_CHUNK = 128


def _latent_body(b_per_w, n_chunks, content_dim, class_dim,
                 img_hbm, cls_hbm, ctab_hbm, ktab_hbm, out_hbm,
                 iidx_v, cidx_v, crow_v, krow_v, csem, ksem):
  wid = lax.axis_index("s") * _NUM_CORES + lax.axis_index("c")
  base = wid * b_per_w
  pltpu.sync_copy(img_hbm.at[wid], iidx_v)
  pltpu.sync_copy(cls_hbm.at[wid], cidx_v)
  copies = []
  for j in range(n_chunks):
    copies.append(pltpu.async_copy(
        ctab_hbm.at[iidx_v.at[j]], crow_v.at[pl.ds(j * _IDX_CHUNK, _IDX_CHUNK)],
        csem))
    copies.append(pltpu.async_copy(
        ktab_hbm.at[cidx_v.at[j]], krow_v.at[pl.ds(j * _IDX_CHUNK, _IDX_CHUNK)],
        ksem))
  for c in copies:
    c.wait()
  pltpu.sync_copy(crow_v, out_hbm.at[pl.ds(base, b_per_w), pl.ds(0, content_dim)])
  pltpu.sync_copy(krow_v, out_hbm.at[pl.ds(base, b_per_w), pl.ds(content_dim, class_dim)])


def kernel(img_id, class_id, content_table, class_table):
  batch = img_id.shape[0]
  content_dim = content_table.shape[1]
  class_dim = class_table.shape[1]
  assert batch % (_NUM_WORKERS * _IDX_CHUNK) == 0
  b_per_w = batch // _NUM_WORKERS
  n_chunks = b_per_w // _IDX_CHUNK

  img_3d = img_id.reshape(_NUM_WORKERS, n_chunks, _IDX_CHUNK)
  cls_3d = class_id.reshape(_NUM_WORKERS, n_chunks, _IDX_CHUNK)

  mesh = plsc.VectorSubcoreMesh(core_axis_name="c", subcore_axis_name="s")
  body = functools.partial(_latent_body, b_per_w, n_chunks, content_dim,
                           class_dim)
  latent = pl.kernel(
      body,
      out_type=jax.ShapeDtypeStruct((batch, content_dim + class_dim),
                                    jnp.float32),
      mesh=mesh,
      scratch_types=[
          pltpu.VMEM((n_chunks, _IDX_CHUNK), jnp.int32),
          pltpu.VMEM((n_chunks, _IDX_CHUNK), jnp.int32),
          pltpu.VMEM((b_per_w, content_dim), jnp.float32),
          pltpu.VMEM((b_per_w, class_dim), jnp.float32),
          pltpu.SemaphoreType.DMA,
          pltpu.SemaphoreType.DMA,
      ],
  )
  return latent(img_3d, cls_3d, content_table, class_table)


# speedup vs baseline: 1.9741x; 1.9741x over previous
import functools

import jax
import jax.numpy as jnp
from jax import lax
from jax.experimental import pallas as pl
from jax.experimental.pallas import tpu as pltpu
from jax.experimental.pallas import tpu_sc as plsc

_NUM_CORES = 2
_NUM_SUBCORES = 16
_NUM_WORKERS = _NUM_CORES * _NUM_SUBCORES
_IDX_CHUNK = 128


def _latent_body(b_per_w, n_chunks, content_dim, class_dim,
                 img_hbm, cls_hbm, ctab_hbm, ktab_hbm, out_hbm,
                 iidx_v, cidx_v, crow_v, krow_v, csem, ksem):
  wid = lax.axis_index("s") * _NUM_CORES + lax.axis_index("c")
  base = wid * b_per_w
  pltpu.sync_copy(img_hbm.at[wid], iidx_v)
  pltpu.sync_copy(cls_hbm.at[wid], cidx_v)
  copies = []
  for j in range(n_chunks):
    copies.append(pltpu.async_copy(
        ctab_hbm.at[iidx_v.at[j]], crow_v.at[pl.ds(j * _IDX_CHUNK, _IDX_CHUNK)],
        csem))
    copies.append(pltpu.async_copy(
        ktab_hbm.at[cidx_v.at[j]], krow_v.at[pl.ds(j * _IDX_CHUNK, _IDX_CHUNK)],
        ksem))
  for c in copies:
    c.wait()
  pltpu.sync_copy(crow_v, out_hbm.at[pl.ds(base, b_per_w), pl.ds(0, content_dim)])
  pltpu.sync_copy(krow_v, out_hbm.at[pl.ds(base, b_per_w), pl.ds(content_dim, class_dim)])


def kernel(img_id, class_id, content_table, class_table):
  batch = img_id.shape[0]
  content_dim = content_table.shape[1]
  class_dim = class_table.shape[1]
  assert batch % (_NUM_WORKERS * _IDX_CHUNK) == 0
  b_per_w = batch // _NUM_WORKERS
  n_chunks = b_per_w // _IDX_CHUNK

  img_3d = img_id.reshape(_NUM_WORKERS, n_chunks, _IDX_CHUNK)
  cls_3d = class_id.reshape(_NUM_WORKERS, n_chunks, _IDX_CHUNK)

  mesh = plsc.VectorSubcoreMesh(core_axis_name="c", subcore_axis_name="s")
  body = functools.partial(_latent_body, b_per_w, n_chunks, content_dim,
                           class_dim)
  latent = pl.kernel(
      body,
      out_type=jax.ShapeDtypeStruct((batch, content_dim + class_dim),
                                    jnp.float32),
      mesh=mesh,
      compiler_params=pltpu.CompilerParams(use_tc_tiling_on_sc=False),
      scratch_types=[
          pltpu.VMEM((n_chunks, _IDX_CHUNK), jnp.int32),
          pltpu.VMEM((n_chunks, _IDX_CHUNK), jnp.int32),
          pltpu.VMEM((b_per_w, content_dim), jnp.float32),
          pltpu.VMEM((b_per_w, class_dim), jnp.float32),
          pltpu.SemaphoreType.DMA,
          pltpu.SemaphoreType.DMA,
      ],
  )
  return latent(img_3d, cls_3d, content_table, class_table)


# speedup vs baseline: 1.9818x; 1.0039x over previous
import functools

import jax
import jax.numpy as jnp
from jax import lax
from jax.experimental import pallas as pl
from jax.experimental.pallas import tpu as pltpu
from jax.experimental.pallas import tpu_sc as plsc

_NUM_CORES = 2
_NUM_SUBCORES = 16
_NUM_WORKERS = _NUM_CORES * _NUM_SUBCORES
_IDX_CHUNK = 128


def _latent_body(b_per_w, n_chunks, content_dim, class_dim,
                 img_hbm, cls_hbm, ctab_hbm, ktab_hbm, out_hbm,
                 iidx_v, cidx_v, crow_v, krow_v, csem, ksem, wsem):
  wid = lax.axis_index("s") * _NUM_CORES + lax.axis_index("c")
  base = wid * b_per_w
  pltpu.sync_copy(img_hbm.at[wid], iidx_v)
  pltpu.sync_copy(cls_hbm.at[wid], cidx_v)
  c_copies = []
  k_copies = []
  for j in range(n_chunks):
    sl = pl.ds(j * _IDX_CHUNK, _IDX_CHUNK)
    c_copies.append(pltpu.async_copy(ctab_hbm.at[iidx_v.at[j]],
                                     crow_v.at[sl], csem.at[j]))
    k_copies.append(pltpu.async_copy(ktab_hbm.at[cidx_v.at[j]],
                                     krow_v.at[sl], ksem.at[j]))
  writes = []
  for j in range(n_chunks):
    sl = pl.ds(j * _IDX_CHUNK, _IDX_CHUNK)
    osl = pl.ds(base + j * _IDX_CHUNK, _IDX_CHUNK)
    c_copies[j].wait()
    writes.append(pltpu.async_copy(
        crow_v.at[sl], out_hbm.at[osl, pl.ds(0, content_dim)], wsem))
    k_copies[j].wait()
    writes.append(pltpu.async_copy(
        krow_v.at[sl], out_hbm.at[osl, pl.ds(content_dim, class_dim)], wsem))
  for w in writes:
    w.wait()


def kernel(img_id, class_id, content_table, class_table):
  batch = img_id.shape[0]
  content_dim = content_table.shape[1]
  class_dim = class_table.shape[1]
  assert batch % (_NUM_WORKERS * _IDX_CHUNK) == 0
  b_per_w = batch // _NUM_WORKERS
  n_chunks = b_per_w // _IDX_CHUNK

  img_3d = img_id.reshape(_NUM_WORKERS, n_chunks, _IDX_CHUNK)
  cls_3d = class_id.reshape(_NUM_WORKERS, n_chunks, _IDX_CHUNK)

  mesh = plsc.VectorSubcoreMesh(core_axis_name="c", subcore_axis_name="s")
  body = functools.partial(_latent_body, b_per_w, n_chunks, content_dim,
                           class_dim)
  latent = pl.kernel(
      body,
      out_type=jax.ShapeDtypeStruct((batch, content_dim + class_dim),
                                    jnp.float32),
      mesh=mesh,
      compiler_params=pltpu.CompilerParams(use_tc_tiling_on_sc=False),
      scratch_types=[
          pltpu.VMEM((n_chunks, _IDX_CHUNK), jnp.int32),
          pltpu.VMEM((n_chunks, _IDX_CHUNK), jnp.int32),
          pltpu.VMEM((b_per_w, content_dim), jnp.float32),
          pltpu.VMEM((b_per_w, class_dim), jnp.float32),
          pltpu.SemaphoreType.DMA((n_chunks,)),
          pltpu.SemaphoreType.DMA((n_chunks,)),
          pltpu.SemaphoreType.DMA,
      ],
  )
  return latent(img_3d, cls_3d, content_table, class_table)
